# baseline (speedup 1.0000x reference)
import functools

import jax
import jax.numpy as jnp
from jax.experimental import pallas as pl
from jax.experimental.pallas import tpu as pltpu

_LOG2E = 1.4426950408889634


def _prepass_body(x_ref, w_ref, asrc_ref, adst_ref, wh_ref, s_ref, d_ref, dmax_ref):
    i = pl.program_id(0)
    wh = jnp.dot(x_ref[...], w_ref[...], preferred_element_type=jnp.float32)
    wh_ref[...] = wh
    s_ref[...] = jnp.dot(wh, asrc_ref[...], preferred_element_type=jnp.float32)
    d = jnp.dot(wh, adst_ref[...], preferred_element_type=jnp.float32)
    d_ref[...] = d
    bmax = jnp.max(d, axis=0, keepdims=True)

    @pl.when(i == 0)
    def _():
        dmax_ref[...] = bmax

    @pl.when(i > 0)
    def _():
        dmax_ref[...] = jnp.maximum(dmax_ref[...], bmax)


def _main_body(adj_ref, s_ref, dt_ref, wh2_ref, dmax_ref, out_ref, acc_ref, srow_ref,
               *, n, h_heads, d_dim, bn, n_col_blocks):
    c = pl.program_id(1)

    @pl.when(c == 0)
    def _():
        s = s_ref[...]
        x = s + dmax_ref[...]
        m = jnp.maximum(x, 0.2 * x)
        srow_ref[:, :h_heads] = (s - m) * _LOG2E
        srow_ref[:, h_heads:2 * h_heads] = (0.2 * s - m) * _LOG2E

    col_ids = c * bn + jax.lax.broadcasted_iota(jnp.int32, (1, bn), 1)
    adjm = jnp.where(col_ids < n, adj_ref[...], 0.0)

    for h in range(h_heads):
        s1 = srow_ref[:, h:h + 1]
        s2 = srow_ref[:, h_heads + h:h_heads + h + 1]
        d1 = dt_ref[h:h + 1, :]
        d2 = dt_ref[h_heads + h:h_heads + h + 1, :]
        t = jnp.maximum(s1 + d1, s2 + d2)
        p = jnp.exp2(t) * adjm
        part = jnp.dot(p, wh2_ref[:, 32 * h:32 * h + 32],
                       preferred_element_type=jnp.float32)

        @pl.when(c == 0)
        def _(part=part, h=h):
            acc_ref[:, 32 * h:32 * h + 32] = part

        @pl.when(c > 0)
        def _(part=part, h=h):
            acc_ref[:, 32 * h:32 * h + 32] += part

    @pl.when(c == n_col_blocks - 1)
    def _():
        for h in range(h_heads):
            y = acc_ref[:, 32 * h:32 * h + d_dim] / \
                acc_ref[:, 32 * h + d_dim:32 * h + d_dim + 1]
            out_ref[:, h * d_dim:(h + 1) * d_dim] = \
                jnp.where(y > 0, y, jnp.exp(y) - 1.0)


def kernel(features, adj, W, a_src, a_dst):
    n, f = features.shape
    h_heads, _, d_dim = W.shape
    hd = h_heads * d_dim

    bm1 = 1000
    r1 = n // bm1
    w_cat = jnp.transpose(W, (1, 0, 2)).reshape(f, hd)
    asrc = jnp.zeros((hd, h_heads), jnp.float32)
    adst = jnp.zeros((hd, h_heads), jnp.float32)
    for h in range(h_heads):
        asrc = asrc.at[h * d_dim:(h + 1) * d_dim, h].set(a_src[h])
        adst = adst.at[h * d_dim:(h + 1) * d_dim, h].set(a_dst[h])

    wh, s, d, dmax = pl.pallas_call(
        _prepass_body,
        grid=(r1,),
        in_specs=[
            pl.BlockSpec((bm1, f), lambda i: (i, 0)),
            pl.BlockSpec((f, hd), lambda i: (0, 0)),
            pl.BlockSpec((hd, h_heads), lambda i: (0, 0)),
            pl.BlockSpec((hd, h_heads), lambda i: (0, 0)),
        ],
        out_specs=[
            pl.BlockSpec((bm1, hd), lambda i: (i, 0)),
            pl.BlockSpec((bm1, h_heads), lambda i: (i, 0)),
            pl.BlockSpec((bm1, h_heads), lambda i: (i, 0)),
            pl.BlockSpec((1, h_heads), lambda i: (0, 0)),
        ],
        out_shape=[
            jax.ShapeDtypeStruct((n, hd), jnp.float32),
            jax.ShapeDtypeStruct((n, h_heads), jnp.float32),
            jax.ShapeDtypeStruct((n, h_heads), jnp.float32),
            jax.ShapeDtypeStruct((1, h_heads), jnp.float32),
        ],
    )(features, w_cat, asrc, adst)

    bm, bn = 1024, 2048
    rr = pl.cdiv(n, bm)
    cc = pl.cdiv(n, bn)
    npad = cc * bn

    dt_pad = jnp.zeros((8, npad), jnp.float32)
    dt_pad = dt_pad.at[:h_heads, :n].set(d.T * _LOG2E)
    dt_pad = dt_pad.at[h_heads:2 * h_heads, :n].set(d.T * (0.2 * _LOG2E))
    wh2_pad = jnp.zeros((npad, 128), jnp.float32)
    for h in range(h_heads):
        wh2_pad = wh2_pad.at[:n, 32 * h:32 * h + d_dim].set(
            wh[:, h * d_dim:(h + 1) * d_dim])
        wh2_pad = wh2_pad.at[:n, 32 * h + d_dim].set(1.0)

    body = functools.partial(_main_body, n=n, h_heads=h_heads, d_dim=d_dim,
                             bn=bn, n_col_blocks=cc)
    out = pl.pallas_call(
        body,
        grid=(rr, cc),
        in_specs=[
            pl.BlockSpec((bm, bn), lambda r, c: (r, c)),
            pl.BlockSpec((bm, h_heads), lambda r, c: (r, 0)),
            pl.BlockSpec((8, bn), lambda r, c: (0, c)),
            pl.BlockSpec((bn, 128), lambda r, c: (c, 0)),
            pl.BlockSpec((1, h_heads), lambda r, c: (0, 0)),
        ],
        out_specs=pl.BlockSpec((bm, hd), lambda r, c: (r, 0)),
        out_shape=jax.ShapeDtypeStruct((n, hd), jnp.float32),
        scratch_shapes=[
            pltpu.VMEM((bm, 128), jnp.float32),
            pltpu.VMEM((bm, 2 * h_heads), jnp.float32),
        ],
        compiler_params=pltpu.CompilerParams(
            dimension_semantics=("parallel", "arbitrary"),
        ),
    )(adj, s, dt_pad, wh2_pad, dmax)
    return out


# speedup vs baseline: 1.0125x; 1.0125x over previous
import functools

import jax
import jax.numpy as jnp
from jax.experimental import pallas as pl
from jax.experimental.pallas import tpu as pltpu

_LOG2E = 1.4426950408889634


def _prepass_body(x_ref, w_ref, asrc_ref, adst_ref, wh_ref, s_ref,
                  dexp_ref, dexp2_ref, dmax_ref):
    i = pl.program_id(0)
    wh = jnp.dot(x_ref[...], w_ref[...], preferred_element_type=jnp.float32)
    wh_ref[...] = wh
    s_ref[...] = jnp.dot(wh, asrc_ref[...], preferred_element_type=jnp.float32)
    d = jnp.dot(wh, adst_ref[...], preferred_element_type=jnp.float32)
    dexp_ref[...] = jnp.exp(d)
    dexp2_ref[...] = jnp.exp(0.2 * d)
    bmax = jnp.max(d, axis=0, keepdims=True)

    @pl.when(i == 0)
    def _():
        dmax_ref[...] = bmax

    @pl.when(i > 0)
    def _():
        dmax_ref[...] = jnp.maximum(dmax_ref[...], bmax)


def _main_body(adj_ref, s_ref, dt_ref, wh2_ref, dmax_ref, out_ref, acc_ref, srow_ref,
               *, n, h_heads, d_dim, bn, n_col_blocks):
    c = pl.program_id(1)

    @pl.when(c == 0)
    def _():
        s = s_ref[...]
        x = s + dmax_ref[...]
        m = jnp.maximum(x, 0.2 * x)
        srow_ref[:, :h_heads] = jnp.exp(s - m)
        srow_ref[:, h_heads:2 * h_heads] = jnp.exp(0.2 * s - m)

    col_ids = c * bn + jax.lax.broadcasted_iota(jnp.int32, (1, bn), 1)
    adjm = jnp.where(col_ids < n, adj_ref[...], 0.0)

    for h in range(h_heads):
        u1 = srow_ref[:, h:h + 1]
        u2 = srow_ref[:, h_heads + h:h_heads + h + 1]
        v1 = dt_ref[h:h + 1, :]
        v2 = dt_ref[h_heads + h:h_heads + h + 1, :]
        p = jnp.maximum(u1 * v1, u2 * v2) * adjm
        part = jnp.dot(p, wh2_ref[:, 32 * h:32 * h + 32],
                       preferred_element_type=jnp.float32)

        @pl.when(c == 0)
        def _(part=part, h=h):
            acc_ref[:, 32 * h:32 * h + 32] = part

        @pl.when(c > 0)
        def _(part=part, h=h):
            acc_ref[:, 32 * h:32 * h + 32] += part

    @pl.when(c == n_col_blocks - 1)
    def _():
        for h in range(h_heads):
            y = acc_ref[:, 32 * h:32 * h + d_dim] / \
                acc_ref[:, 32 * h + d_dim:32 * h + d_dim + 1]
            out_ref[:, h * d_dim:(h + 1) * d_dim] = \
                jnp.where(y > 0, y, jnp.exp(y) - 1.0)


def kernel(features, adj, W, a_src, a_dst):
    n, f = features.shape
    h_heads, _, d_dim = W.shape
    hd = h_heads * d_dim

    bm1 = 1000
    r1 = n // bm1
    w_cat = jnp.transpose(W, (1, 0, 2)).reshape(f, hd)
    asrc = jnp.zeros((hd, h_heads), jnp.float32)
    adst = jnp.zeros((hd, h_heads), jnp.float32)
    for h in range(h_heads):
        asrc = asrc.at[h * d_dim:(h + 1) * d_dim, h].set(a_src[h])
        adst = adst.at[h * d_dim:(h + 1) * d_dim, h].set(a_dst[h])

    wh, s, dexp, dexp2, dmax = pl.pallas_call(
        _prepass_body,
        grid=(r1,),
        in_specs=[
            pl.BlockSpec((bm1, f), lambda i: (i, 0)),
            pl.BlockSpec((f, hd), lambda i: (0, 0)),
            pl.BlockSpec((hd, h_heads), lambda i: (0, 0)),
            pl.BlockSpec((hd, h_heads), lambda i: (0, 0)),
        ],
        out_specs=[
            pl.BlockSpec((bm1, hd), lambda i: (i, 0)),
            pl.BlockSpec((bm1, h_heads), lambda i: (i, 0)),
            pl.BlockSpec((bm1, h_heads), lambda i: (i, 0)),
            pl.BlockSpec((bm1, h_heads), lambda i: (i, 0)),
            pl.BlockSpec((1, h_heads), lambda i: (0, 0)),
        ],
        out_shape=[
            jax.ShapeDtypeStruct((n, hd), jnp.float32),
            jax.ShapeDtypeStruct((n, h_heads), jnp.float32),
            jax.ShapeDtypeStruct((n, h_heads), jnp.float32),
            jax.ShapeDtypeStruct((n, h_heads), jnp.float32),
            jax.ShapeDtypeStruct((1, h_heads), jnp.float32),
        ],
    )(features, w_cat, asrc, adst)

    bm, bn = 1024, 2048
    rr = pl.cdiv(n, bm)
    cc = pl.cdiv(n, bn)
    npad = cc * bn

    dt_pad = jnp.zeros((8, npad), jnp.float32)
    dt_pad = dt_pad.at[:h_heads, :n].set(dexp.T)
    dt_pad = dt_pad.at[h_heads:2 * h_heads, :n].set(dexp2.T)
    wh2_pad = jnp.zeros((npad, 128), jnp.float32)
    for h in range(h_heads):
        wh2_pad = wh2_pad.at[:n, 32 * h:32 * h + d_dim].set(
            wh[:, h * d_dim:(h + 1) * d_dim])
        wh2_pad = wh2_pad.at[:n, 32 * h + d_dim].set(1.0)

    body = functools.partial(_main_body, n=n, h_heads=h_heads, d_dim=d_dim,
                             bn=bn, n_col_blocks=cc)
    out = pl.pallas_call(
        body,
        grid=(rr, cc),
        in_specs=[
            pl.BlockSpec((bm, bn), lambda r, c: (r, c)),
            pl.BlockSpec((bm, h_heads), lambda r, c: (r, 0)),
            pl.BlockSpec((8, bn), lambda r, c: (0, c)),
            pl.BlockSpec((bn, 128), lambda r, c: (c, 0)),
            pl.BlockSpec((1, h_heads), lambda r, c: (0, 0)),
        ],
        out_specs=pl.BlockSpec((bm, hd), lambda r, c: (r, 0)),
        out_shape=jax.ShapeDtypeStruct((n, hd), jnp.float32),
        scratch_shapes=[
            pltpu.VMEM((bm, 128), jnp.float32),
            pltpu.VMEM((bm, 2 * h_heads), jnp.float32),
        ],
        compiler_params=pltpu.CompilerParams(
            dimension_semantics=("parallel", "arbitrary"),
        ),
    )(adj, s, dt_pad, wh2_pad, dmax)
    return out


# speedup vs baseline: 1.1461x; 1.1319x over previous
import functools

import jax
import jax.numpy as jnp
from jax.experimental import pallas as pl
from jax.experimental.pallas import tpu as pltpu

_LOG2E = 1.4426950408889634


def _prepass_body(x_ref, w_ref, asrc_ref, adst_ref, wh_ref, s_ref,
                  dexp_ref, dexp2_ref, dmax_ref):
    i = pl.program_id(0)
    wh = jnp.dot(x_ref[...], w_ref[...], preferred_element_type=jnp.float32)
    wh_ref[...] = wh
    s_ref[...] = jnp.dot(wh, asrc_ref[...], preferred_element_type=jnp.float32)
    d = jnp.dot(wh, adst_ref[...], preferred_element_type=jnp.float32)
    dexp_ref[...] = jnp.exp(d)
    dexp2_ref[...] = jnp.exp(0.2 * d)
    bmax = jnp.max(d, axis=0, keepdims=True)

    @pl.when(i == 0)
    def _():
        dmax_ref[...] = bmax

    @pl.when(i > 0)
    def _():
        dmax_ref[...] = jnp.maximum(dmax_ref[...], bmax)


def _main_body(*refs, n, h_heads, d_dim, bn, k_chunks, n_col_steps):
    adj_refs = refs[:k_chunks]
    dt_refs = refs[k_chunks:2 * k_chunks]
    wh2_refs = refs[2 * k_chunks:3 * k_chunks]
    s_ref, dmax_ref, out_ref, acc_ref, srow_ref = refs[3 * k_chunks:]
    c = pl.program_id(1)

    @pl.when(c == 0)
    def _():
        s = s_ref[...]
        x = s + dmax_ref[...]
        m = jnp.maximum(x, 0.2 * x)
        srow_ref[:, :h_heads] = jnp.exp(s - m)
        srow_ref[:, h_heads:2 * h_heads] = jnp.exp(0.2 * s - m)

    for k in range(k_chunks):
        col_ids = ((c * k_chunks + k) * bn
                   + jax.lax.broadcasted_iota(jnp.int32, (1, bn), 1))
        adjm = jnp.where(col_ids < n, adj_refs[k][...], 0.0)

        for h in range(h_heads):
            u1 = srow_ref[:, h:h + 1]
            u2 = srow_ref[:, h_heads + h:h_heads + h + 1]
            v1 = dt_refs[k][h:h + 1, :]
            v2 = dt_refs[k][h_heads + h:h_heads + h + 1, :]
            p = jnp.maximum(u1 * v1, u2 * v2) * adjm
            part = jnp.dot(p, wh2_refs[k][:, 32 * h:32 * h + 32],
                           preferred_element_type=jnp.float32)

            if k == 0:
                @pl.when(c == 0)
                def _(part=part, h=h):
                    acc_ref[:, 32 * h:32 * h + 32] = part

                @pl.when(c > 0)
                def _(part=part, h=h):
                    acc_ref[:, 32 * h:32 * h + 32] += part
            else:
                acc_ref[:, 32 * h:32 * h + 32] += part

    @pl.when(c == n_col_steps - 1)
    def _():
        for h in range(h_heads):
            y = acc_ref[:, 32 * h:32 * h + d_dim] / \
                acc_ref[:, 32 * h + d_dim:32 * h + d_dim + 1]
            out_ref[:, h * d_dim:(h + 1) * d_dim] = \
                jnp.where(y > 0, y, jnp.exp(y) - 1.0)


def kernel(features, adj, W, a_src, a_dst):
    n, f = features.shape
    h_heads, _, d_dim = W.shape
    hd = h_heads * d_dim

    bm1 = 1000
    r1 = n // bm1
    w_cat = jnp.transpose(W, (1, 0, 2)).reshape(f, hd)
    asrc = jnp.zeros((hd, h_heads), jnp.float32)
    adst = jnp.zeros((hd, h_heads), jnp.float32)
    for h in range(h_heads):
        asrc = asrc.at[h * d_dim:(h + 1) * d_dim, h].set(a_src[h])
        adst = adst.at[h * d_dim:(h + 1) * d_dim, h].set(a_dst[h])

    wh, s, dexp, dexp2, dmax = pl.pallas_call(
        _prepass_body,
        grid=(r1,),
        in_specs=[
            pl.BlockSpec((bm1, f), lambda i: (i, 0)),
            pl.BlockSpec((f, hd), lambda i: (0, 0)),
            pl.BlockSpec((hd, h_heads), lambda i: (0, 0)),
            pl.BlockSpec((hd, h_heads), lambda i: (0, 0)),
        ],
        out_specs=[
            pl.BlockSpec((bm1, hd), lambda i: (i, 0)),
            pl.BlockSpec((bm1, h_heads), lambda i: (i, 0)),
            pl.BlockSpec((bm1, h_heads), lambda i: (i, 0)),
            pl.BlockSpec((bm1, h_heads), lambda i: (i, 0)),
            pl.BlockSpec((1, h_heads), lambda i: (0, 0)),
        ],
        out_shape=[
            jax.ShapeDtypeStruct((n, hd), jnp.float32),
            jax.ShapeDtypeStruct((n, h_heads), jnp.float32),
            jax.ShapeDtypeStruct((n, h_heads), jnp.float32),
            jax.ShapeDtypeStruct((n, h_heads), jnp.float32),
            jax.ShapeDtypeStruct((1, h_heads), jnp.float32),
        ],
    )(features, w_cat, asrc, adst)

    bm, bn, kc = 512, 1024, 5
    rr = pl.cdiv(n, bm)
    cc2 = pl.cdiv(n, bn * kc)
    npad = cc2 * kc * bn

    dt_pad = jnp.zeros((8, npad), jnp.float32)
    dt_pad = dt_pad.at[:h_heads, :n].set(dexp.T)
    dt_pad = dt_pad.at[h_heads:2 * h_heads, :n].set(dexp2.T)
    wh2_pad = jnp.zeros((npad, 128), jnp.float32)
    for h in range(h_heads):
        wh2_pad = wh2_pad.at[:n, 32 * h:32 * h + d_dim].set(
            wh[:, h * d_dim:(h + 1) * d_dim])
        wh2_pad = wh2_pad.at[:n, 32 * h + d_dim].set(1.0)

    body = functools.partial(_main_body, n=n, h_heads=h_heads, d_dim=d_dim,
                             bn=bn, k_chunks=kc, n_col_steps=cc2)
    adj_specs = [pl.BlockSpec((bm, bn), lambda r, c, k=k: (r, c * kc + k))
                 for k in range(kc)]
    dt_specs = [pl.BlockSpec((8, bn), lambda r, c, k=k: (0, c * kc + k))
                for k in range(kc)]
    wh2_specs = [pl.BlockSpec((bn, 128), lambda r, c, k=k: (c * kc + k, 0))
                 for k in range(kc)]
    out = pl.pallas_call(
        body,
        grid=(rr, cc2),
        in_specs=adj_specs + dt_specs + wh2_specs + [
            pl.BlockSpec((bm, h_heads), lambda r, c: (r, 0)),
            pl.BlockSpec((1, h_heads), lambda r, c: (0, 0)),
        ],
        out_specs=pl.BlockSpec((bm, hd), lambda r, c: (r, 0)),
        out_shape=jax.ShapeDtypeStruct((n, hd), jnp.float32),
        scratch_shapes=[
            pltpu.VMEM((bm, 128), jnp.float32),
            pltpu.VMEM((bm, 2 * h_heads), jnp.float32),
        ],
        compiler_params=pltpu.CompilerParams(
            dimension_semantics=("parallel", "arbitrary"),
        ),
    )(*([adj] * kc), *([dt_pad] * kc), *([wh2_pad] * kc), s, dmax)
    return out


# speedup vs baseline: 1.3362x; 1.1659x over previous
import functools

import jax
import jax.numpy as jnp
from jax.experimental import pallas as pl
from jax.experimental.pallas import tpu as pltpu

_LOG2E = 1.4426950408889634


def _prepass_body(x_ref, w_ref, asrc_ref, adst_ref, wh_ref, s_ref,
                  dexp_ref, dexp2_ref, dmax_ref):
    i = pl.program_id(0)
    wh = jnp.dot(x_ref[...], w_ref[...], preferred_element_type=jnp.float32)
    wh_ref[...] = wh
    s_ref[...] = jnp.dot(wh, asrc_ref[...], preferred_element_type=jnp.float32)
    d = jnp.dot(wh, adst_ref[...], preferred_element_type=jnp.float32)
    dexp_ref[...] = jnp.exp(d)
    dexp2_ref[...] = jnp.exp(0.2 * d)
    bmax = jnp.max(d, axis=0, keepdims=True)

    @pl.when(i == 0)
    def _():
        dmax_ref[...] = bmax

    @pl.when(i > 0)
    def _():
        dmax_ref[...] = jnp.maximum(dmax_ref[...], bmax)


def _main_body(*refs, n, h_heads, d_dim, bn, k_chunks, n_col_steps):
    adj_refs = refs[:k_chunks]
    dt_refs = refs[k_chunks:2 * k_chunks]
    wh2_refs = refs[2 * k_chunks:3 * k_chunks]
    s_ref, dmax_ref, out_ref, acc_ref, srow_ref = refs[3 * k_chunks:]
    c = pl.program_id(1)

    @pl.when(c == 0)
    def _():
        s = s_ref[...]
        x = s + dmax_ref[...]
        m = jnp.maximum(x, 0.2 * x)
        srow_ref[:, :h_heads] = jnp.exp(s - m)
        srow_ref[:, h_heads:2 * h_heads] = jnp.exp(0.2 * s - m)

    for k in range(k_chunks):
        col_ids = ((c * k_chunks + k) * bn
                   + jax.lax.broadcasted_iota(jnp.int32, (1, bn), 1))
        adjm = jnp.where(col_ids < n, adj_refs[k][...], 0.0)

        for h in range(h_heads):
            u1 = srow_ref[:, h:h + 1]
            u2 = srow_ref[:, h_heads + h:h_heads + h + 1]
            v1 = dt_refs[k][h:h + 1, :]
            v2 = dt_refs[k][h_heads + h:h_heads + h + 1, :]
            p = jnp.maximum(u1 * v1, u2 * v2) * adjm
            part = jnp.dot(p, wh2_refs[k][:, 32 * h:32 * h + 32],
                           preferred_element_type=jnp.float32)

            if k == 0:
                @pl.when(c == 0)
                def _(part=part, h=h):
                    acc_ref[:, 32 * h:32 * h + 32] = part

                @pl.when(c > 0)
                def _(part=part, h=h):
                    acc_ref[:, 32 * h:32 * h + 32] += part
            else:
                acc_ref[:, 32 * h:32 * h + 32] += part

    @pl.when(c == n_col_steps - 1)
    def _():
        for h in range(h_heads):
            y = acc_ref[:, 32 * h:32 * h + d_dim] / \
                acc_ref[:, 32 * h + d_dim:32 * h + d_dim + 1]
            out_ref[:, h * d_dim:(h + 1) * d_dim] = \
                jnp.where(y > 0, y, jnp.exp(y) - 1.0)


def kernel(features, adj, W, a_src, a_dst):
    n, f = features.shape
    h_heads, _, d_dim = W.shape
    hd = h_heads * d_dim

    bm1 = 1000
    r1 = n // bm1
    w_cat = jnp.transpose(W, (1, 0, 2)).reshape(f, hd)
    asrc = jnp.zeros((hd, h_heads), jnp.float32)
    adst = jnp.zeros((hd, h_heads), jnp.float32)
    for h in range(h_heads):
        asrc = asrc.at[h * d_dim:(h + 1) * d_dim, h].set(a_src[h])
        adst = adst.at[h * d_dim:(h + 1) * d_dim, h].set(a_dst[h])

    wh, s, dexp, dexp2, dmax = pl.pallas_call(
        _prepass_body,
        grid=(r1,),
        in_specs=[
            pl.BlockSpec((bm1, f), lambda i: (i, 0)),
            pl.BlockSpec((f, hd), lambda i: (0, 0)),
            pl.BlockSpec((hd, h_heads), lambda i: (0, 0)),
            pl.BlockSpec((hd, h_heads), lambda i: (0, 0)),
        ],
        out_specs=[
            pl.BlockSpec((bm1, hd), lambda i: (i, 0)),
            pl.BlockSpec((bm1, h_heads), lambda i: (i, 0)),
            pl.BlockSpec((bm1, h_heads), lambda i: (i, 0)),
            pl.BlockSpec((bm1, h_heads), lambda i: (i, 0)),
            pl.BlockSpec((1, h_heads), lambda i: (0, 0)),
        ],
        out_shape=[
            jax.ShapeDtypeStruct((n, hd), jnp.float32),
            jax.ShapeDtypeStruct((n, h_heads), jnp.float32),
            jax.ShapeDtypeStruct((n, h_heads), jnp.float32),
            jax.ShapeDtypeStruct((n, h_heads), jnp.float32),
            jax.ShapeDtypeStruct((1, h_heads), jnp.float32),
        ],
    )(features, w_cat, asrc, adst)

    bm, bn, kc = 256, 10240, 1
    rr = pl.cdiv(n, bm)
    cc2 = pl.cdiv(n, bn * kc)
    npad = cc2 * kc * bn

    dt_pad = jnp.zeros((8, npad), jnp.float32)
    dt_pad = dt_pad.at[:h_heads, :n].set(dexp.T)
    dt_pad = dt_pad.at[h_heads:2 * h_heads, :n].set(dexp2.T)
    wh2_pad = jnp.zeros((npad, 128), jnp.float32)
    for h in range(h_heads):
        wh2_pad = wh2_pad.at[:n, 32 * h:32 * h + d_dim].set(
            wh[:, h * d_dim:(h + 1) * d_dim])
        wh2_pad = wh2_pad.at[:n, 32 * h + d_dim].set(1.0)

    body = functools.partial(_main_body, n=n, h_heads=h_heads, d_dim=d_dim,
                             bn=bn, k_chunks=kc, n_col_steps=cc2)
    adj_specs = [pl.BlockSpec((bm, bn), lambda r, c, k=k: (r, c * kc + k))
                 for k in range(kc)]
    dt_specs = [pl.BlockSpec((8, bn), lambda r, c, k=k: (0, c * kc + k))
                for k in range(kc)]
    wh2_specs = [pl.BlockSpec((bn, 128), lambda r, c, k=k: (c * kc + k, 0))
                 for k in range(kc)]
    out = pl.pallas_call(
        body,
        grid=(rr, cc2),
        in_specs=adj_specs + dt_specs + wh2_specs + [
            pl.BlockSpec((bm, h_heads), lambda r, c: (r, 0)),
            pl.BlockSpec((1, h_heads), lambda r, c: (0, 0)),
        ],
        out_specs=pl.BlockSpec((bm, hd), lambda r, c: (r, 0)),
        out_shape=jax.ShapeDtypeStruct((n, hd), jnp.float32),
        scratch_shapes=[
            pltpu.VMEM((bm, 128), jnp.float32),
            pltpu.VMEM((bm, 2 * h_heads), jnp.float32),
        ],
        compiler_params=pltpu.CompilerParams(
            dimension_semantics=("parallel", "arbitrary"),
        ),
    )(*([adj] * kc), *([dt_pad] * kc), *([wh2_pad] * kc), s, dmax)
    return out
